# baseline (speedup 1.0000x reference)
import functools

import jax
import jax.numpy as jnp
from jax import lax
from jax.experimental import pallas as pl
from jax.experimental.pallas import tpu as pltpu, tpu_sc as plsc

NUM_CORES = 2
NUM_SUBCORES = 16
LANES = 16
NW = NUM_CORES * NUM_SUBCORES

NUM_USERS = 1000000
BATCH = 16384
DIM = 16
TLANES = 128
B_PER_W = BATCH // NW
CHUNKS = B_PER_W // LANES
JROWS = 128
NBUF = 3


def _body(users_hbm, items_hbm, itab_hbm, utab_hbm, out_hbm,
          uidx_v, iidx_v, itab_v, buf_v, out_v, sems):
    wid = lax.axis_index("s") * NUM_CORES + lax.axis_index("c")
    base = wid * (B_PER_W // JROWS)

    pltpu.sync_copy(users_hbm.at[pl.ds(base, B_PER_W // JROWS)], uidx_v)
    pltpu.sync_copy(items_hbm.at[pl.ds(base, B_PER_W // JROWS)], iidx_v)
    pltpu.sync_copy(itab_hbm, itab_v)

    iota = lax.iota(jnp.int32, LANES)

    def load_chunk_idx(k):
        j = jax.lax.shift_right_logical(k, 3)
        sl = pl.ds(jnp.bitwise_and(k, 7) * LANES, LANES)
        return j, sl

    def fire(k, slot):
        j, sl = load_chunk_idx(k)
        v = uidx_v[j, sl]
        for h in range(8):
            for t in range(LANES):
                c = jax.lax.shift_right_logical(v[t], 7) * TLANES
                c = pl.multiple_of(c, TLANES)
                pltpu.async_copy(
                    utab_hbm.at[pl.ds(h * 2, 2), pl.ds(c, TLANES)],
                    buf_v.at[slot, t, pl.ds(h * 2, 2)], sems.at[slot])

    def drain(slot):
        for t in range(LANES):
            for h in range(2):
                pltpu.make_async_copy(
                    utab_hbm.at[pl.ds(h * 8, 8), pl.ds(0, TLANES)],
                    buf_v.at[slot, t, pl.ds(h * 8, 8)], sems.at[slot]).wait()

    for k in range(NBUF):
        fire(k, k)

    def step(k, _):
        slot = lax.rem(k, NBUF)
        drain(slot)

        j, sl = load_chunk_idx(k)
        rmod = jnp.bitwise_and(uidx_v[j, sl], TLANES - 1)
        items_g = iidx_v[j, sl]
        acc = jnp.zeros((LANES,), jnp.float32)
        slotv = jnp.full((LANES,), slot, jnp.int32)
        for d in range(DIM):
            dsplat = jnp.full((LANES,), d, jnp.int32)
            ucol = plsc.load_gather(buf_v, [slotv, iota, dsplat, rmod])
            icol = plsc.load_gather(itab_v, [dsplat, items_g])
            acc = acc + ucol * icol
        out_v[j, sl] = acc

        @pl.when(k + NBUF < CHUNKS)
        def _():
            fire(k + NBUF, slot)
        return 0

    lax.fori_loop(0, CHUNKS, step, 0)

    pltpu.sync_copy(out_v, out_hbm.at[pl.ds(base, B_PER_W // JROWS)])


@jax.jit
def kernel(users, items, item_table, user_table):
    users2 = users.reshape(BATCH // JROWS, JROWS).astype(jnp.int32)
    items2 = items.reshape(BATCH // JROWS, JROWS).astype(jnp.int32)
    itab2 = item_table.T
    utab_t = user_table.T

    mesh = plsc.VectorSubcoreMesh(core_axis_name="c", subcore_axis_name="s")
    run = functools.partial(
        pl.kernel,
        mesh=mesh,
        out_type=jax.ShapeDtypeStruct((BATCH // JROWS, JROWS), jnp.float32),
        scratch_types=[
            pltpu.VMEM((B_PER_W // JROWS, JROWS), jnp.int32),
            pltpu.VMEM((B_PER_W // JROWS, JROWS), jnp.int32),
            pltpu.VMEM((DIM, 64), jnp.float32),
            pltpu.VMEM((NBUF, LANES, DIM, TLANES), jnp.float32),
            pltpu.VMEM((B_PER_W // JROWS, JROWS), jnp.float32),
            pltpu.SemaphoreType.DMA((NBUF,)),
        ],
        compiler_params=pltpu.CompilerParams(needs_layout_passes=False),
    )(_body)
    out2 = run(users2, items2, itab2, utab_t)
    return out2.reshape(BATCH)


# speedup vs baseline: 1.0139x; 1.0139x over previous
import functools

import jax
import jax.numpy as jnp
from jax import lax
from jax.experimental import pallas as pl
from jax.experimental.pallas import tpu as pltpu, tpu_sc as plsc

NUM_CORES = 2
NUM_SUBCORES = 16
LANES = 16
NW = NUM_CORES * NUM_SUBCORES

NUM_USERS = 1000000
BATCH = 16384
DIM = 16
TLANES = 128
B_PER_W = BATCH // NW
CHUNKS = B_PER_W // LANES
JROWS = 128
NBUF = 3


def _body(users_hbm, items_hbm, itab_hbm, utab_hbm, out_hbm,
          uidx_v, iidx_v, itab_v, buf_v, out_v, sems):
    wid = lax.axis_index("s") * NUM_CORES + lax.axis_index("c")
    base = wid * (B_PER_W // JROWS)

    pltpu.sync_copy(users_hbm.at[pl.ds(base, B_PER_W // JROWS)], uidx_v)
    pltpu.sync_copy(items_hbm.at[pl.ds(base, B_PER_W // JROWS)], iidx_v)
    pltpu.sync_copy(itab_hbm, itab_v)

    iota = lax.iota(jnp.int32, LANES)

    def load_chunk_idx(k):
        j = jax.lax.shift_right_logical(k, 3)
        sl = pl.ds(jnp.bitwise_and(k, 7) * LANES, LANES)
        return j, sl

    def fire(k, slot):
        j, sl = load_chunk_idx(k)
        v = uidx_v[j, sl]
        for h in range(4):
            for t in range(LANES):
                c = jax.lax.shift_right_logical(v[t], 7) * TLANES
                c = pl.multiple_of(c, TLANES)
                pltpu.async_copy(
                    utab_hbm.at[pl.ds(h * 4, 4), pl.ds(c, TLANES)],
                    buf_v.at[slot, pl.ds(t * DIM + h * 4, 4)], sems.at[slot])

    def drain(slot):
        for h in range(2):
            pltpu.make_async_copy(
                out_hbm,
                buf_v.at[slot, pl.ds(h * JROWS, JROWS)], sems.at[slot]).wait()

    for k in range(NBUF):
        fire(k, k)

    def step(k, _):
        slot = lax.rem(k, NBUF)
        drain(slot)

        j, sl = load_chunk_idx(k)
        rmod = jnp.bitwise_and(uidx_v[j, sl], TLANES - 1)
        items_g = iidx_v[j, sl]
        acc = jnp.zeros((LANES,), jnp.float32)
        slotv = jnp.full((LANES,), slot, jnp.int32)
        rowbase = iota * DIM
        for d in range(DIM):
            dsplat = jnp.full((LANES,), d, jnp.int32)
            ucol = plsc.load_gather(buf_v, [slotv, rowbase + d, rmod])
            icol = plsc.load_gather(itab_v, [dsplat, items_g])
            acc = acc + ucol * icol
        out_v[j, sl] = acc

        @pl.when(k + NBUF < CHUNKS)
        def _():
            fire(k + NBUF, slot)
        return 0

    lax.fori_loop(0, CHUNKS, step, 0)

    pltpu.sync_copy(out_v, out_hbm.at[pl.ds(base, B_PER_W // JROWS)])


@jax.jit
def kernel(users, items, item_table, user_table):
    users2 = users.reshape(BATCH // JROWS, JROWS).astype(jnp.int32)
    items2 = items.reshape(BATCH // JROWS, JROWS).astype(jnp.int32)
    itab2 = item_table.T
    utab_t = user_table.T

    mesh = plsc.VectorSubcoreMesh(core_axis_name="c", subcore_axis_name="s")
    run = functools.partial(
        pl.kernel,
        mesh=mesh,
        out_type=jax.ShapeDtypeStruct((BATCH // JROWS, JROWS), jnp.float32),
        scratch_types=[
            pltpu.VMEM((B_PER_W // JROWS, JROWS), jnp.int32),
            pltpu.VMEM((B_PER_W // JROWS, JROWS), jnp.int32),
            pltpu.VMEM((DIM, 64), jnp.float32),
            pltpu.VMEM((NBUF, LANES * DIM, TLANES), jnp.float32),
            pltpu.VMEM((B_PER_W // JROWS, JROWS), jnp.float32),
            pltpu.SemaphoreType.DMA((NBUF,)),
        ],
        compiler_params=pltpu.CompilerParams(needs_layout_passes=False),
    )(_body)
    out2 = run(users2, items2, itab2, utab_t)
    return out2.reshape(BATCH)


# speedup vs baseline: 1.0286x; 1.0144x over previous
import functools

import jax
import jax.numpy as jnp
from jax import lax
from jax.experimental import pallas as pl
from jax.experimental.pallas import tpu as pltpu, tpu_sc as plsc

NUM_CORES = 2
NUM_SUBCORES = 16
LANES = 16
NW = NUM_CORES * NUM_SUBCORES

NUM_USERS = 1000000
BATCH = 16384
DIM = 16
TLANES = 128
B_PER_W = BATCH // NW
CHUNKS = B_PER_W // LANES
JROWS = 128
NBUF = 3


def _body(users_hbm, items_hbm, itab_hbm, utab_hbm, out_hbm,
          uidx_v, iidx_v, itab_v, buf_v, out_v, sems, isem):
    wid = lax.axis_index("s") * NUM_CORES + lax.axis_index("c")
    base = wid * (B_PER_W // JROWS)

    pltpu.sync_copy(users_hbm.at[pl.ds(base, B_PER_W // JROWS)], uidx_v)
    h_items = pltpu.async_copy(
        items_hbm.at[pl.ds(base, B_PER_W // JROWS)], iidx_v, isem)
    h_itab = pltpu.async_copy(itab_hbm, itab_v, isem)

    iota = lax.iota(jnp.int32, LANES)

    def load_chunk_idx(k):
        j = jax.lax.shift_right_logical(k, 3)
        sl = pl.ds(jnp.bitwise_and(k, 7) * LANES, LANES)
        return j, sl

    def fire(k, slot):
        j, sl = load_chunk_idx(k)
        v = uidx_v[j, sl]
        for h in range(4):
            for t in range(LANES):
                c = jax.lax.shift_right_logical(v[t], 7) * TLANES
                c = pl.multiple_of(c, TLANES)
                pltpu.async_copy(
                    utab_hbm.at[pl.ds(h * 4, 4), pl.ds(c, TLANES)],
                    buf_v.at[slot, pl.ds(t * DIM + h * 4, 4)], sems.at[slot])

    def drain(slot):
        for h in range(2):
            pltpu.make_async_copy(
                out_hbm,
                buf_v.at[slot, pl.ds(h * JROWS, JROWS)], sems.at[slot]).wait()

    for k in range(NBUF):
        fire(k, k)
    h_items.wait()
    h_itab.wait()

    def step(k, _):
        slot = lax.rem(k, NBUF)
        drain(slot)

        j, sl = load_chunk_idx(k)
        rmod = jnp.bitwise_and(uidx_v[j, sl], TLANES - 1)
        items_g = iidx_v[j, sl]
        acc = jnp.zeros((LANES,), jnp.float32)
        slotv = jnp.full((LANES,), slot, jnp.int32)
        rowbase = iota * DIM
        for d in range(DIM):
            dsplat = jnp.full((LANES,), d, jnp.int32)
            ucol = plsc.load_gather(buf_v, [slotv, rowbase + d, rmod])
            icol = plsc.load_gather(itab_v, [dsplat, items_g])
            acc = acc + ucol * icol
        out_v[j, sl] = acc

        @pl.when(k + NBUF < CHUNKS)
        def _():
            fire(k + NBUF, slot)
        return 0

    lax.fori_loop(0, CHUNKS, step, 0)

    pltpu.sync_copy(out_v, out_hbm.at[pl.ds(base, B_PER_W // JROWS)])


@jax.jit
def kernel(users, items, item_table, user_table):
    users2 = users.reshape(BATCH // JROWS, JROWS).astype(jnp.int32)
    items2 = items.reshape(BATCH // JROWS, JROWS).astype(jnp.int32)
    itab2 = item_table.T
    utab_t = user_table.T

    mesh = plsc.VectorSubcoreMesh(core_axis_name="c", subcore_axis_name="s")
    run = functools.partial(
        pl.kernel,
        mesh=mesh,
        out_type=jax.ShapeDtypeStruct((BATCH // JROWS, JROWS), jnp.float32),
        scratch_types=[
            pltpu.VMEM((B_PER_W // JROWS, JROWS), jnp.int32),
            pltpu.VMEM((B_PER_W // JROWS, JROWS), jnp.int32),
            pltpu.VMEM((DIM, 64), jnp.float32),
            pltpu.VMEM((NBUF, LANES * DIM, TLANES), jnp.float32),
            pltpu.VMEM((B_PER_W // JROWS, JROWS), jnp.float32),
            pltpu.SemaphoreType.DMA((NBUF,)),
            pltpu.SemaphoreType.DMA,
        ],
        compiler_params=pltpu.CompilerParams(needs_layout_passes=False),
    )(_body)
    out2 = run(users2, items2, itab2, utab_t)
    return out2.reshape(BATCH)


# speedup vs baseline: 1.0328x; 1.0042x over previous
import functools

import jax
import jax.numpy as jnp
from jax import lax
from jax.experimental import pallas as pl
from jax.experimental.pallas import tpu as pltpu, tpu_sc as plsc

NUM_CORES = 2
NUM_SUBCORES = 16
LANES = 16
NW = NUM_CORES * NUM_SUBCORES

NUM_USERS = 1000000
BATCH = 16384
DIM = 16
TLANES = 128
B_PER_W = BATCH // NW
CHUNKS = B_PER_W // LANES
JROWS = 128
NBUF = 3


def _body(users_hbm, items_hbm, itab_hbm, utab_hbm, out_hbm,
          uidx_v, iidx_v, itab_v, buf_v, out_v, sems, isem):
    wid = lax.axis_index("s") * NUM_CORES + lax.axis_index("c")
    base = wid * (B_PER_W // JROWS)

    pltpu.sync_copy(users_hbm.at[pl.ds(base, B_PER_W // JROWS)], uidx_v)
    h_items = pltpu.async_copy(
        items_hbm.at[pl.ds(base, B_PER_W // JROWS)], iidx_v, isem)
    h_itab = pltpu.async_copy(itab_hbm, itab_v, isem)

    iota = lax.iota(jnp.int32, LANES)

    def load_chunk_idx(k):
        j = jax.lax.shift_right_logical(k, 3)
        sl = pl.ds(jnp.bitwise_and(k, 7) * LANES, LANES)
        return j, sl

    def fire(k, slot):
        j, sl = load_chunk_idx(k)
        v = uidx_v[j, sl]
        cs = [pl.multiple_of(jax.lax.shift_right_logical(v[t], 7) * TLANES,
                             TLANES) for t in range(LANES)]
        for h in range(4):
            for t in range(LANES):
                pltpu.async_copy(
                    utab_hbm.at[pl.ds(h * 4, 4), pl.ds(cs[t], TLANES)],
                    buf_v.at[slot, pl.ds(t * DIM + h * 4, 4)], sems.at[slot])

    def drain(slot):
        for h in range(2):
            pltpu.make_async_copy(
                out_hbm,
                buf_v.at[slot, pl.ds(h * JROWS, JROWS)], sems.at[slot]).wait()

    for k in range(NBUF):
        fire(k, k)
    h_items.wait()
    h_itab.wait()

    def step(k, _):
        slot = lax.rem(k, NBUF)
        drain(slot)

        j, sl = load_chunk_idx(k)
        rmod = jnp.bitwise_and(uidx_v[j, sl], TLANES - 1)
        items_g = iidx_v[j, sl]
        acc = jnp.zeros((LANES,), jnp.float32)
        slotv = jnp.full((LANES,), slot, jnp.int32)
        rowbase = iota * DIM
        for d in range(DIM):
            dsplat = jnp.full((LANES,), d, jnp.int32)
            ucol = plsc.load_gather(buf_v, [slotv, rowbase + d, rmod])
            icol = plsc.load_gather(itab_v, [dsplat, items_g])
            acc = acc + ucol * icol
        out_v[j, sl] = acc

        @pl.when(k + NBUF < CHUNKS)
        def _():
            fire(k + NBUF, slot)
        return 0

    lax.fori_loop(0, CHUNKS, step, 0)

    pltpu.sync_copy(out_v, out_hbm.at[pl.ds(base, B_PER_W // JROWS)])


@jax.jit
def kernel(users, items, item_table, user_table):
    users2 = users.reshape(BATCH // JROWS, JROWS).astype(jnp.int32)
    items2 = items.reshape(BATCH // JROWS, JROWS).astype(jnp.int32)
    itab2 = item_table.T
    utab_t = user_table.T

    mesh = plsc.VectorSubcoreMesh(core_axis_name="c", subcore_axis_name="s")
    run = functools.partial(
        pl.kernel,
        mesh=mesh,
        out_type=jax.ShapeDtypeStruct((BATCH // JROWS, JROWS), jnp.float32),
        scratch_types=[
            pltpu.VMEM((B_PER_W // JROWS, JROWS), jnp.int32),
            pltpu.VMEM((B_PER_W // JROWS, JROWS), jnp.int32),
            pltpu.VMEM((DIM, 64), jnp.float32),
            pltpu.VMEM((NBUF, LANES * DIM, TLANES), jnp.float32),
            pltpu.VMEM((B_PER_W // JROWS, JROWS), jnp.float32),
            pltpu.SemaphoreType.DMA((NBUF,)),
            pltpu.SemaphoreType.DMA,
        ],
        compiler_params=pltpu.CompilerParams(needs_layout_passes=False),
    )(_body)
    out2 = run(users2, items2, itab2, utab_t)
    return out2.reshape(BATCH)
